# baseline (speedup 1.0000x reference)
import functools

import jax
import jax.numpy as jnp
from jax import lax
from jax.experimental import pallas as pl
from jax.experimental.pallas import tpu as pltpu
from jax.experimental.pallas import tpu_sc as plsc

_LANES = 16
_GROUP = 16
_NBUF = 3


@functools.cache
def _build_gather(B, V, D):
    info = plsc.get_sparse_core_info()
    nw = info.num_cores * info.num_subcores
    assert B % (nw * _GROUP) == 0, (B, nw)
    assert 128 % D == 0 and V % 8 == 0
    b_per_w = B // nw
    n_groups = b_per_w // _GROUP

    mesh = plsc.VectorSubcoreMesh(core_axis_name="c", subcore_axis_name="s")

    @functools.partial(
        pl.kernel,
        mesh=mesh,
        out_type=jax.ShapeDtypeStruct((D, B), jnp.float32),
        scratch_types=[
            pltpu.VMEM((b_per_w,), jnp.int32),
            pltpu.VMEM((b_per_w,), jnp.int32),
            pltpu.VMEM((_NBUF, _GROUP, 8, D), jnp.float32),
            pltpu.VMEM((D, b_per_w), jnp.float32),
            pltpu.SemaphoreType.DMA,
            pltpu.SemaphoreType.DMA,
            pltpu.SemaphoreType.DMA,
        ],
        compiler_params=pltpu.CompilerParams(needs_layout_passes=False),
    )
    def gather_kernel(
        table_hbm,
        idxp_hbm,
        idx_hbm,
        out_hbm,
        idxp_v,
        idx_v,
        slab_v,
        out_v,
        sem0,
        sem1,
        sem2,
    ):
        wid = lax.axis_index("s") * info.num_cores + lax.axis_index("c")
        base = pl.multiple_of(wid * b_per_w, b_per_w)
        table3 = table_hbm
        pltpu.sync_copy(idxp_hbm.at[pl.ds(base, b_per_w)], idxp_v)
        pltpu.sync_copy(idx_hbm.at[pl.ds(base, b_per_w)], idx_v)

        lane_iota = lax.iota(jnp.int32, _LANES)

        def fire(g, buf, sem):
            for h in range(_GROUP // _LANES):
                p16 = idxp_v[
                    pl.ds(pl.multiple_of(g * _GROUP + h * _LANES, _LANES), _LANES)
                ]
                for k in range(_LANES):
                    pltpu.async_copy(
                        table3.at[p16[k]], slab_v.at[buf, h * _LANES + k], sem
                    )

        def drain(buf, sem):
            pltpu.make_async_copy(
                table3.at[pl.ds(0, _GROUP)], slab_v.at[buf], sem
            ).wait()

        def extract(g, buf):
            zero16 = jnp.zeros((_LANES,), jnp.int32)
            for h in range(_GROUP // _LANES):
                gbase = pl.multiple_of(g * _GROUP + h * _LANES, _LANES)
                raw16 = idx_v[pl.ds(gbase, _LANES)]
                j16 = lax.rem(raw16, 8)
                t16 = gbase + lane_iota
                s16 = h * _LANES + lane_iota

                @pl.loop(0, D, unroll=8)
                def per_feature(f):
                    f16 = zero16 + f
                    vals = plsc.load_gather(slab_v.at[buf], [s16, j16, f16])
                    plsc.store_scatter(out_v, [f16, t16], vals)

        sems = [sem0, sem1, sem2]
        for g in range(n_groups):
            b = g % _NBUF
            if g >= _NBUF:
                drain(b, sems[b])
                extract(g - _NBUF, b)
            fire(g, b, sems[b])
        for g in range(n_groups - _NBUF, n_groups):
            b = g % _NBUF
            drain(b, sems[b])
            extract(g, b)

        pltpu.sync_copy(out_v, out_hbm.at[:, pl.ds(base, b_per_w)])

    return gather_kernel


def kernel(weight, input_idx, dim):
    V, D = weight.shape
    B = input_idx.shape[0]
    idx = (input_idx + dim).astype(jnp.int32)
    table3 = weight.reshape(V // 8, 8, D)
    outT = _build_gather(B, V, D)(table3, idx // 8, idx)
    return outT.T


# speedup vs baseline: 1.0243x; 1.0243x over previous
import functools

import jax
import jax.numpy as jnp
from jax import lax
from jax.experimental import pallas as pl
from jax.experimental.pallas import tpu as pltpu
from jax.experimental.pallas import tpu_sc as plsc

_LANES = 16
_GROUP = 32


@functools.cache
def _build_gather(B, V, D):
    info = plsc.get_sparse_core_info()
    nw = info.num_cores * info.num_subcores
    assert B % (nw * _GROUP) == 0, (B, nw)
    assert 128 % D == 0 and V % 8 == 0
    b_per_w = B // nw
    n_groups = b_per_w // _GROUP

    mesh = plsc.VectorSubcoreMesh(core_axis_name="c", subcore_axis_name="s")

    @functools.partial(
        pl.kernel,
        mesh=mesh,
        out_type=jax.ShapeDtypeStruct((D, B), jnp.float32),
        scratch_types=[
            pltpu.VMEM((b_per_w,), jnp.int32),
            pltpu.VMEM((b_per_w,), jnp.int32),
            pltpu.VMEM((2, _GROUP, 8, D), jnp.float32),
            pltpu.VMEM((D, b_per_w), jnp.float32),
            pltpu.SemaphoreType.DMA,
            pltpu.SemaphoreType.DMA,
        ],
        compiler_params=pltpu.CompilerParams(needs_layout_passes=False),
    )
    def gather_kernel(
        table_hbm,
        idxp_hbm,
        idx_hbm,
        out_hbm,
        idxp_v,
        idx_v,
        slab_v,
        out_v,
        sem0,
        sem1,
    ):
        wid = lax.axis_index("s") * info.num_cores + lax.axis_index("c")
        base = pl.multiple_of(wid * b_per_w, b_per_w)
        table3 = table_hbm
        pltpu.sync_copy(idxp_hbm.at[pl.ds(base, b_per_w)], idxp_v)
        pltpu.sync_copy(idx_hbm.at[pl.ds(base, b_per_w)], idx_v)

        lane_iota = lax.iota(jnp.int32, _LANES)

        def fire(g, buf, sem):
            for h in range(_GROUP // _LANES):
                p16 = idxp_v[
                    pl.ds(pl.multiple_of(g * _GROUP + h * _LANES, _LANES), _LANES)
                ]
                for k in range(_LANES):
                    pltpu.async_copy(
                        table3.at[p16[k]], slab_v.at[buf, h * _LANES + k], sem
                    )

        def drain(buf, sem):
            pltpu.make_async_copy(
                table3.at[pl.ds(0, _GROUP)], slab_v.at[buf], sem
            ).wait()

        def extract(g, buf):
            zero16 = jnp.zeros((_LANES,), jnp.int32)
            for h in range(_GROUP // _LANES):
                gbase = pl.multiple_of(g * _GROUP + h * _LANES, _LANES)
                raw16 = idx_v[pl.ds(gbase, _LANES)]
                j16 = lax.rem(raw16, 8)
                t16 = gbase + lane_iota
                s16 = h * _LANES + lane_iota

                @pl.loop(0, D, unroll=8)
                def per_feature(f):
                    f16 = zero16 + f
                    vals = plsc.load_gather(slab_v.at[buf], [s16, j16, f16])
                    plsc.store_scatter(out_v, [f16, t16], vals)

        assert n_groups % 2 == 0

        @pl.loop(0, n_groups, step=2)
        def per_pair(g):
            fire(g, 0, sem0)

            @pl.when(g >= 2)
            def _():
                drain(1, sem1)
                extract(g - 1, 1)

            fire(g + 1, 1, sem1)
            drain(0, sem0)
            extract(g, 0)

        drain(1, sem1)
        extract(n_groups - 1, 1)

        pltpu.sync_copy(out_v, out_hbm.at[:, pl.ds(base, b_per_w)])

    return gather_kernel


def kernel(weight, input_idx, dim):
    V, D = weight.shape
    B = input_idx.shape[0]
    idx = (input_idx + dim).astype(jnp.int32)
    table3 = weight.reshape(V // 8, 8, D)
    outT = _build_gather(B, V, D)(table3, idx // 8, idx)
    return outT.T


# speedup vs baseline: 1.0253x; 1.0010x over previous
import functools

import jax
import jax.numpy as jnp
from jax import lax
from jax.experimental import pallas as pl
from jax.experimental.pallas import tpu as pltpu
from jax.experimental.pallas import tpu_sc as plsc

_LANES = 16
_GROUP = 32


@functools.cache
def _build_gather(B, V, D):
    info = plsc.get_sparse_core_info()
    nw = info.num_cores * info.num_subcores
    assert B % (nw * _GROUP) == 0, (B, nw)
    assert 128 % D == 0 and V % 8 == 0
    b_per_w = B // nw
    n_groups = b_per_w // _GROUP

    mesh = plsc.VectorSubcoreMesh(core_axis_name="c", subcore_axis_name="s")

    @functools.partial(
        pl.kernel,
        mesh=mesh,
        out_type=jax.ShapeDtypeStruct((D, B), jnp.float32),
        scratch_types=[
            pltpu.VMEM((b_per_w,), jnp.int32),
            pltpu.VMEM((2, _GROUP, 8, D), jnp.float32),
            pltpu.VMEM((D, b_per_w), jnp.float32),
            pltpu.SemaphoreType.DMA,
            pltpu.SemaphoreType.DMA,
        ],
        compiler_params=pltpu.CompilerParams(needs_layout_passes=False),
    )
    def gather_kernel(
        table_hbm,
        idx_hbm,
        out_hbm,
        idx_v,
        slab_v,
        out_v,
        sem0,
        sem1,
    ):
        wid = lax.axis_index("s") * info.num_cores + lax.axis_index("c")
        base = pl.multiple_of(wid * b_per_w, b_per_w)
        table3 = table_hbm
        pltpu.sync_copy(idx_hbm.at[pl.ds(base, b_per_w)], idx_v)

        lane_iota = lax.iota(jnp.int32, _LANES)

        def fire(g, buf, sem):
            for h in range(_GROUP // _LANES):
                p16 = (
                    idx_v[
                        pl.ds(
                            pl.multiple_of(g * _GROUP + h * _LANES, _LANES),
                            _LANES,
                        )
                    ]
                    >> 3
                )
                for k in range(_LANES):
                    pltpu.async_copy(
                        table3.at[p16[k]], slab_v.at[buf, h * _LANES + k], sem
                    )

        def drain(buf, sem):
            pltpu.make_async_copy(
                table3.at[pl.ds(0, _GROUP)], slab_v.at[buf], sem
            ).wait()

        def extract(g, buf):
            zero16 = jnp.zeros((_LANES,), jnp.int32)
            for h in range(_GROUP // _LANES):
                gbase = pl.multiple_of(g * _GROUP + h * _LANES, _LANES)
                raw16 = idx_v[pl.ds(gbase, _LANES)]
                j16 = lax.rem(raw16, 8)
                t16 = gbase + lane_iota
                s16 = h * _LANES + lane_iota

                @pl.loop(0, D, unroll=8)
                def per_feature(f):
                    f16 = zero16 + f
                    vals = plsc.load_gather(slab_v.at[buf], [s16, j16, f16])
                    plsc.store_scatter(out_v, [f16, t16], vals)

        assert n_groups % 2 == 0

        @pl.loop(0, n_groups, step=2)
        def per_pair(g):
            fire(g, 0, sem0)

            @pl.when(g >= 2)
            def _():
                drain(1, sem1)
                extract(g - 1, 1)

            fire(g + 1, 1, sem1)
            drain(0, sem0)
            extract(g, 0)

        drain(1, sem1)
        extract(n_groups - 1, 1)

        pltpu.sync_copy(out_v, out_hbm.at[:, pl.ds(base, b_per_w)])

    return gather_kernel


def kernel(weight, input_idx, dim):
    V, D = weight.shape
    B = input_idx.shape[0]
    idx = (input_idx + dim).astype(jnp.int32)
    table3 = weight.reshape(V // 8, 8, D)
    outT = _build_gather(B, V, D)(table3, idx)
    return outT.T
